# initial kernel scaffold (unmeasured)
import jax
import jax.numpy as jnp
from jax import lax
from jax.experimental import pallas as pl
from jax.experimental.pallas import tpu as pltpu

N_DEV = 4
M, K, N = 4096, 4096, 8192
CHUNK = M // N_DEV
TILE = 256
N_TILES = CHUNK // TILE

_HBM = pltpu.MemorySpace.HBM
_SMEM = pltpu.MemorySpace.SMEM
_VMEM = pltpu.MemorySpace.VMEM


def _body(partial_ref, scale_ref, out_ref, comm_ref,
          a_tile, b_tile, f_tile,
          rs_send, rs_recv, ag_send, ag_recv, loc_sem):
    my = lax.axis_index("i")
    right = lax.rem(my + 1, N_DEV)
    left = lax.rem(my + N_DEV - 1, N_DEV)

    barrier_sem = pltpu.get_barrier_semaphore()
    for nbr in (left, right):
        pl.semaphore_signal(barrier_sem, inc=1, device_id=(nbr,),
                            device_id_type=pl.DeviceIdType.MESH)
    pl.semaphore_wait(barrier_sem, 2)

    def copy(src, dst, sem):
        cp = pltpu.make_async_copy(src, dst, sem)
        cp.start()
        return cp

    for s in range(N_DEV - 1):
        c_send = lax.rem(my - s + N_DEV, N_DEV)
        c_recv = lax.rem(my - s - 1 + N_DEV, N_DEV)
        if s == 0:
            src = partial_ref.at[pl.ds(c_send * CHUNK, CHUNK), :]
        else:
            src = comm_ref.at[s - 1]
        rdma = pltpu.make_async_remote_copy(
            src_ref=src,
            dst_ref=comm_ref.at[s],
            send_sem=rs_send.at[s],
            recv_sem=rs_recv.at[s],
            device_id=(right,),
            device_id_type=pl.DeviceIdType.MESH,
        )
        rdma.start()
        rdma.wait()

        row0 = c_recv * CHUNK
        if s < N_DEV - 2:
            for t in range(N_TILES):
                ca = copy(comm_ref.at[s, pl.ds(t * TILE, TILE), :],
                          a_tile, loc_sem.at[0])
                cb = copy(partial_ref.at[pl.ds(row0 + t * TILE, TILE), :],
                          b_tile, loc_sem.at[1])
                ca.wait()
                cb.wait()
                a_tile[...] = a_tile[...] + b_tile[...]
                copy(a_tile, comm_ref.at[s, pl.ds(t * TILE, TILE), :],
                     loc_sem.at[0]).wait()
        else:
            for t in range(N_TILES):
                ca = copy(comm_ref.at[s, pl.ds(t * TILE, TILE), :],
                          a_tile, loc_sem.at[0])
                cb = copy(partial_ref.at[pl.ds(row0 + t * TILE, TILE), :],
                          b_tile, loc_sem.at[1])
                ca.wait()
                cb.wait()
                acc = a_tile[...] + b_tile[...]
                y = acc.astype(jnp.float32) * scale_ref[0, 0]
                z = jnp.clip(y, -60.0, 60.0)
                f_tile[...] = y / (1.0 + jnp.exp(-z))
                copy(f_tile, out_ref.at[pl.ds(row0 + t * TILE, TILE), :],
                     loc_sem.at[0]).wait()

    for t in range(N_DEV - 1):
        c = lax.rem(my + 1 - t + N_DEV, N_DEV)
        rows = pl.ds(c * CHUNK, CHUNK)
        rdma = pltpu.make_async_remote_copy(
            src_ref=out_ref.at[rows, :],
            dst_ref=out_ref.at[rows, :],
            send_sem=ag_send.at[t],
            recv_sem=ag_recv.at[t],
            device_id=(right,),
            device_id_type=pl.DeviceIdType.MESH,
        )
        rdma.start()
        rdma.wait()


def kernel(x, w_mat, scale_x, scale_w):
    partial = lax.dot_general(
        x, w_mat,
        dimension_numbers=(((1,), (0,)), ((), ())),
        preferred_element_type=jnp.int32,
    )
    scale = (scale_x[0] * scale_w[0]).reshape(1, 1)

    return pl.pallas_call(
        _body,
        out_shape=jax.ShapeDtypeStruct((M, N), jnp.float32),
        in_specs=[
            pl.BlockSpec(memory_space=_HBM),
            pl.BlockSpec(memory_space=_SMEM),
        ],
        out_specs=pl.BlockSpec(memory_space=_HBM),
        scratch_shapes=[
            _HBM(shape=(N_DEV - 1, CHUNK, N), dtype=jnp.int32),
            _VMEM(shape=(TILE, N), dtype=jnp.int32),
            _VMEM(shape=(TILE, N), dtype=jnp.int32),
            _VMEM(shape=(TILE, N), dtype=jnp.float32),
            pltpu.SemaphoreType.DMA((N_DEV - 1,)),
            pltpu.SemaphoreType.DMA((N_DEV - 1,)),
            pltpu.SemaphoreType.DMA((N_DEV - 1,)),
            pltpu.SemaphoreType.DMA((N_DEV - 1,)),
            pltpu.SemaphoreType.DMA((2,)),
        ],
        compiler_params=pltpu.CompilerParams(collective_id=0),
    )(partial, scale)


# baseline (device time: 2476780 ns/iter reference)
import jax
import jax.numpy as jnp
from jax import lax
from jax.experimental import pallas as pl
from jax.experimental.pallas import tpu as pltpu

N_DEV = 4
M, K, N = 4096, 4096, 8192
CHUNK = M // N_DEV
TILE = 256
N_TILES = CHUNK // TILE

_HBM = pltpu.MemorySpace.HBM
_SMEM = pltpu.MemorySpace.SMEM
_VMEM = pltpu.MemorySpace.VMEM


def _body(partial_ref, scale_ref, out_ref, comm_ref,
          a_tile, b_tile, f_tile,
          rs_send, rs_recv, ag_send, ag_recv, loc_sem):
    my = lax.axis_index("i")
    right = lax.rem(my + 1, N_DEV)
    left = lax.rem(my + N_DEV - 1, N_DEV)

    barrier_sem = pltpu.get_barrier_semaphore()
    for nbr in (left, right):
        pl.semaphore_signal(barrier_sem, inc=1, device_id=(nbr,),
                            device_id_type=pl.DeviceIdType.MESH)
    pl.semaphore_wait(barrier_sem, 2)

    def copy(src, dst, sem):
        cp = pltpu.make_async_copy(src, dst, sem)
        cp.start()
        return cp

    for s in range(N_DEV - 1):
        c_send = lax.rem(my - s + N_DEV, N_DEV)
        c_recv = lax.rem(my - s - 1 + N_DEV, N_DEV)
        if s == 0:
            src = partial_ref.at[pl.ds(c_send * CHUNK, CHUNK), :]
        else:
            src = comm_ref.at[s - 1]
        rdma = pltpu.make_async_remote_copy(
            src_ref=src,
            dst_ref=comm_ref.at[s],
            send_sem=rs_send.at[s],
            recv_sem=rs_recv.at[s],
            device_id=(right,),
            device_id_type=pl.DeviceIdType.MESH,
        )
        rdma.start()
        rdma.wait()

        row0 = c_recv * CHUNK
        if s < N_DEV - 2:
            for t in range(N_TILES):
                ca = copy(comm_ref.at[s, pl.ds(t * TILE, TILE), :],
                          a_tile, loc_sem.at[0])
                cb = copy(partial_ref.at[pl.ds(row0 + t * TILE, TILE), :],
                          b_tile, loc_sem.at[1])
                ca.wait()
                cb.wait()
                a_tile[...] = a_tile[...] + b_tile[...]
                copy(a_tile, comm_ref.at[s, pl.ds(t * TILE, TILE), :],
                     loc_sem.at[0]).wait()
        else:
            for t in range(N_TILES):
                ca = copy(comm_ref.at[s, pl.ds(t * TILE, TILE), :],
                          a_tile, loc_sem.at[0])
                cb = copy(partial_ref.at[pl.ds(row0 + t * TILE, TILE), :],
                          b_tile, loc_sem.at[1])
                ca.wait()
                cb.wait()
                acc = a_tile[...] + b_tile[...]
                y = acc.astype(jnp.float32) * scale_ref[0, 0]
                z = jnp.clip(y, -60.0, 60.0)
                f_tile[...] = y / (1.0 + jnp.exp(-z))
                copy(f_tile, out_ref.at[pl.ds(row0 + t * TILE, TILE), :],
                     loc_sem.at[0]).wait()

    for t in range(N_DEV - 1):
        c = lax.rem(my + 1 - t + N_DEV, N_DEV)
        rows = pl.ds(c * CHUNK, CHUNK)
        rdma = pltpu.make_async_remote_copy(
            src_ref=out_ref.at[rows, :],
            dst_ref=out_ref.at[rows, :],
            send_sem=ag_send.at[t],
            recv_sem=ag_recv.at[t],
            device_id=(right,),
            device_id_type=pl.DeviceIdType.MESH,
        )
        rdma.start()
        rdma.wait()


def kernel(x, w_mat, scale_x, scale_w):
    partial = lax.dot_general(
        x, w_mat,
        dimension_numbers=(((1,), (0,)), ((), ())),
        preferred_element_type=jnp.int32,
    )
    scale = (scale_x[0] * scale_w[0]).reshape(1, 1)

    out, _ = pl.pallas_call(
        _body,
        out_shape=[
            jax.ShapeDtypeStruct((M, N), jnp.float32),
            jax.ShapeDtypeStruct((N_DEV - 1, CHUNK, N), jnp.int32),
        ],
        in_specs=[
            pl.BlockSpec(memory_space=_HBM),
            pl.BlockSpec(memory_space=_SMEM),
        ],
        out_specs=[
            pl.BlockSpec(memory_space=_HBM),
            pl.BlockSpec(memory_space=_HBM),
        ],
        scratch_shapes=[
            _VMEM(shape=(TILE, N), dtype=jnp.int32),
            _VMEM(shape=(TILE, N), dtype=jnp.int32),
            _VMEM(shape=(TILE, N), dtype=jnp.float32),
            pltpu.SemaphoreType.DMA((N_DEV - 1,)),
            pltpu.SemaphoreType.DMA((N_DEV - 1,)),
            pltpu.SemaphoreType.DMA((N_DEV - 1,)),
            pltpu.SemaphoreType.DMA((N_DEV - 1,)),
            pltpu.SemaphoreType.DMA((2,)),
        ],
        compiler_params=pltpu.CompilerParams(collective_id=0),
    )(partial, scale)
    return out


# device time: 1420927 ns/iter; 1.7431x vs baseline; 1.7431x over previous
import jax
import jax.numpy as jnp
from jax import lax
from jax.experimental import pallas as pl
from jax.experimental.pallas import tpu as pltpu

N_DEV = 4
M, K, N = 4096, 4096, 8192
CHUNK = M // N_DEV
H = N // 2
TILE = 256
N_TILES = CHUNK // TILE

_HBM = pltpu.MemorySpace.HBM
_SMEM = pltpu.MemorySpace.SMEM
_VMEM = pltpu.MemorySpace.VMEM


def _body(partial_ref, scale_ref, out_ref, comm_ref,
          a_tile, b_tile, f_tile,
          rs_send, rs_recv, ag_send, ag_recv, loc_sem):
    my = lax.axis_index("i")
    right = lax.rem(my + 1, N_DEV)
    left = lax.rem(my + N_DEV - 1, N_DEV)

    barrier_sem = pltpu.get_barrier_semaphore()
    for nbr in (left, right):
        pl.semaphore_signal(barrier_sem, inc=1, device_id=(nbr,),
                            device_id_type=pl.DeviceIdType.MESH)
    pl.semaphore_wait(barrier_sem, 2)

    def copy(src, dst, sem):
        cp = pltpu.make_async_copy(src, dst, sem)
        cp.start()
        return cp

    def rs_chunks(d, s):
        if d == 0:
            return lax.rem(my - s + N_DEV, N_DEV), lax.rem(my - s - 1 + N_DEV, N_DEV)
        return lax.rem(my + s, N_DEV), lax.rem(my + s + 1, N_DEV)

    for s in range(N_DEV - 1):
        rdmas = []
        recvs = []
        for d, peer in ((0, right), (1, left)):
            col0 = d * H
            c_send, c_recv = rs_chunks(d, s)
            recvs.append(c_recv)
            if s == 0:
                src = partial_ref.at[pl.ds(c_send * CHUNK, CHUNK), pl.ds(col0, H)]
            else:
                src = comm_ref.at[s - 1, :, pl.ds(col0, H)]
            rdma = pltpu.make_async_remote_copy(
                src_ref=src,
                dst_ref=comm_ref.at[s, :, pl.ds(col0, H)],
                send_sem=rs_send.at[d, s],
                recv_sem=rs_recv.at[d, s],
                device_id=(peer,),
                device_id_type=pl.DeviceIdType.MESH,
            )
            rdma.start()
            rdmas.append(rdma)
        for rdma in rdmas:
            rdma.wait()

        for d in (0, 1):
            col0 = d * H
            row0 = recvs[d] * CHUNK
            for t in range(N_TILES):
                ca = copy(comm_ref.at[s, pl.ds(t * TILE, TILE), pl.ds(col0, H)],
                          a_tile, loc_sem.at[0])
                cb = copy(partial_ref.at[pl.ds(row0 + t * TILE, TILE),
                                         pl.ds(col0, H)],
                          b_tile, loc_sem.at[1])
                ca.wait()
                cb.wait()
                if s < N_DEV - 2:
                    a_tile[...] = a_tile[...] + b_tile[...]
                    copy(a_tile,
                         comm_ref.at[s, pl.ds(t * TILE, TILE), pl.ds(col0, H)],
                         loc_sem.at[0]).wait()
                else:
                    acc = a_tile[...] + b_tile[...]
                    y = acc.astype(jnp.float32) * scale_ref[0, 0]
                    z = jnp.clip(y, -60.0, 60.0)
                    f_tile[...] = y / (1.0 + jnp.exp(-z))
                    copy(f_tile,
                         out_ref.at[pl.ds(row0 + t * TILE, TILE), pl.ds(col0, H)],
                         loc_sem.at[0]).wait()

    for t in range(N_DEV - 1):
        rdmas = []
        for d, peer in ((0, right), (1, left)):
            col0 = d * H
            if d == 0:
                c = lax.rem(my + 1 - t + N_DEV, N_DEV)
            else:
                c = lax.rem(my - 1 + t + N_DEV, N_DEV)
            rows = pl.ds(c * CHUNK, CHUNK)
            rdma = pltpu.make_async_remote_copy(
                src_ref=out_ref.at[rows, pl.ds(col0, H)],
                dst_ref=out_ref.at[rows, pl.ds(col0, H)],
                send_sem=ag_send.at[d, t],
                recv_sem=ag_recv.at[d, t],
                device_id=(peer,),
                device_id_type=pl.DeviceIdType.MESH,
            )
            rdma.start()
            rdmas.append(rdma)
        for rdma in rdmas:
            rdma.wait()


def kernel(x, w_mat, scale_x, scale_w):
    partial = lax.dot_general(
        x, w_mat,
        dimension_numbers=(((1,), (0,)), ((), ())),
        preferred_element_type=jnp.int32,
    )
    scale = (scale_x[0] * scale_w[0]).reshape(1, 1)

    out, _ = pl.pallas_call(
        _body,
        out_shape=[
            jax.ShapeDtypeStruct((M, N), jnp.float32),
            jax.ShapeDtypeStruct((N_DEV - 1, CHUNK, N), jnp.int32),
        ],
        in_specs=[
            pl.BlockSpec(memory_space=_HBM),
            pl.BlockSpec(memory_space=_SMEM),
        ],
        out_specs=[
            pl.BlockSpec(memory_space=_HBM),
            pl.BlockSpec(memory_space=_HBM),
        ],
        scratch_shapes=[
            _VMEM(shape=(TILE, H), dtype=jnp.int32),
            _VMEM(shape=(TILE, H), dtype=jnp.int32),
            _VMEM(shape=(TILE, H), dtype=jnp.float32),
            pltpu.SemaphoreType.DMA((2, N_DEV - 1)),
            pltpu.SemaphoreType.DMA((2, N_DEV - 1)),
            pltpu.SemaphoreType.DMA((2, N_DEV - 1)),
            pltpu.SemaphoreType.DMA((2, N_DEV - 1)),
            pltpu.SemaphoreType.DMA((2,)),
        ],
        compiler_params=pltpu.CompilerParams(collective_id=0),
    )(partial, scale)
    return out


# device time: 1263568 ns/iter; 1.9601x vs baseline; 1.1245x over previous
import jax
import jax.numpy as jnp
from jax import lax
from jax.experimental import pallas as pl
from jax.experimental.pallas import tpu as pltpu

N_DEV = 4
M, K, N = 4096, 4096, 8192
CHUNK = M // N_DEV
H = N // 2
SUB = 4
TILE = CHUNK // SUB

_HBM = pltpu.MemorySpace.HBM
_SMEM = pltpu.MemorySpace.SMEM
_VMEM = pltpu.MemorySpace.VMEM


def _body(partial_ref, scale_ref, out_ref, comm_ref,
          a_tile, b_tile, f_tile,
          rs_send, rs_recv, ag_send, ag_recv, loc_sem):
    my = lax.axis_index("i")
    right = lax.rem(my + 1, N_DEV)
    left = lax.rem(my + N_DEV - 1, N_DEV)
    peers = (right, left)

    barrier_sem = pltpu.get_barrier_semaphore()
    for nbr in peers:
        pl.semaphore_signal(barrier_sem, inc=1, device_id=(nbr,),
                            device_id_type=pl.DeviceIdType.MESH)
    pl.semaphore_wait(barrier_sem, 2)

    def copy(src, dst, sem):
        cp = pltpu.make_async_copy(src, dst, sem)
        cp.start()
        return cp

    def rs_chunks(d, s):
        if d == 0:
            return (lax.rem(my - s + N_DEV, N_DEV),
                    lax.rem(my - s - 1 + N_DEV, N_DEV))
        return lax.rem(my + s, N_DEV), lax.rem(my + s + 1, N_DEV)

    def make_rs(d, s, q):
        col = pl.ds(d * H, H)
        c_send, _ = rs_chunks(d, s)
        if s == 0:
            src = partial_ref.at[pl.ds(c_send * CHUNK + q * TILE, TILE), col]
        else:
            src = comm_ref.at[s - 1, pl.ds(q * TILE, TILE), col]
        return pltpu.make_async_remote_copy(
            src_ref=src,
            dst_ref=comm_ref.at[s, pl.ds(q * TILE, TILE), col],
            send_sem=rs_send.at[d, s, q],
            recv_sem=rs_recv.at[d, s, q],
            device_id=(peers[d],),
            device_id_type=pl.DeviceIdType.MESH,
        )

    def make_ag(d, t, q):
        col = pl.ds(d * H, H)
        if d == 0:
            c = lax.rem(my + 1 - t + N_DEV, N_DEV)
        else:
            c = lax.rem(my - 1 + t + N_DEV, N_DEV)
        rows = pl.ds(c * CHUNK + q * TILE, TILE)
        return pltpu.make_async_remote_copy(
            src_ref=out_ref.at[rows, col],
            dst_ref=out_ref.at[rows, col],
            send_sem=ag_send.at[d, t, q],
            recv_sem=ag_recv.at[d, t, q],
            device_id=(peers[d],),
            device_id_type=pl.DeviceIdType.MESH,
        )

    rs_rdma = {}
    ag_rdma = {}

    for q in range(SUB):
        for d in (0, 1):
            r = make_rs(d, 0, q)
            r.start()
            rs_rdma[(d, 0, q)] = r

    for s in range(N_DEV - 1):
        for q in range(SUB):
            for d in (0, 1):
                rs_rdma[(d, s, q)].wait_recv()
                col = pl.ds(d * H, H)
                row0 = rs_chunks(d, s)[1] * CHUNK + q * TILE
                ca = copy(comm_ref.at[s, pl.ds(q * TILE, TILE), col],
                          a_tile, loc_sem.at[0])
                cb = copy(partial_ref.at[pl.ds(row0, TILE), col],
                          b_tile, loc_sem.at[1])
                ca.wait()
                cb.wait()
                if s < N_DEV - 2:
                    a_tile[...] = a_tile[...] + b_tile[...]
                    copy(a_tile, comm_ref.at[s, pl.ds(q * TILE, TILE), col],
                         loc_sem.at[0]).wait()
                    r = make_rs(d, s + 1, q)
                    r.start()
                    rs_rdma[(d, s + 1, q)] = r
                else:
                    acc = a_tile[...] + b_tile[...]
                    y = acc.astype(jnp.float32) * scale_ref[0, 0]
                    z = jnp.clip(y, -60.0, 60.0)
                    f_tile[...] = y / (1.0 + jnp.exp(-z))
                    copy(f_tile, out_ref.at[pl.ds(row0, TILE), col],
                         loc_sem.at[0]).wait()
                    r = make_ag(d, 0, q)
                    r.start()
                    ag_rdma[(d, 0, q)] = r

    for t in range(N_DEV - 1):
        for q in range(SUB):
            for d in (0, 1):
                ag_rdma[(d, t, q)].wait_recv()
                if t < N_DEV - 2:
                    r = make_ag(d, t + 1, q)
                    r.start()
                    ag_rdma[(d, t + 1, q)] = r

    for r in rs_rdma.values():
        r.wait_send()
    for r in ag_rdma.values():
        r.wait_send()


def kernel(x, w_mat, scale_x, scale_w):
    partial = lax.dot_general(
        x, w_mat,
        dimension_numbers=(((1,), (0,)), ((), ())),
        preferred_element_type=jnp.int32,
    )
    scale = (scale_x[0] * scale_w[0]).reshape(1, 1)

    out, _ = pl.pallas_call(
        _body,
        out_shape=[
            jax.ShapeDtypeStruct((M, N), jnp.float32),
            jax.ShapeDtypeStruct((N_DEV - 1, CHUNK, N), jnp.int32),
        ],
        in_specs=[
            pl.BlockSpec(memory_space=_HBM),
            pl.BlockSpec(memory_space=_SMEM),
        ],
        out_specs=[
            pl.BlockSpec(memory_space=_HBM),
            pl.BlockSpec(memory_space=_HBM),
        ],
        scratch_shapes=[
            _VMEM(shape=(TILE, H), dtype=jnp.int32),
            _VMEM(shape=(TILE, H), dtype=jnp.int32),
            _VMEM(shape=(TILE, H), dtype=jnp.float32),
            pltpu.SemaphoreType.DMA((2, N_DEV - 1, SUB)),
            pltpu.SemaphoreType.DMA((2, N_DEV - 1, SUB)),
            pltpu.SemaphoreType.DMA((2, N_DEV - 1, SUB)),
            pltpu.SemaphoreType.DMA((2, N_DEV - 1, SUB)),
            pltpu.SemaphoreType.DMA((2,)),
        ],
        compiler_params=pltpu.CompilerParams(collective_id=0),
    )(partial, scale)
    return out


# device time: 1263030 ns/iter; 1.9610x vs baseline; 1.0004x over previous
import jax
import jax.numpy as jnp
from jax import lax
from jax.experimental import pallas as pl
from jax.experimental.pallas import tpu as pltpu

N_DEV = 4
M, K, N = 4096, 4096, 8192
CHUNK = M // N_DEV
H = N // 2
SUB = 4
TILE = CHUNK // SUB

_HBM = pltpu.MemorySpace.HBM
_SMEM = pltpu.MemorySpace.SMEM
_VMEM = pltpu.MemorySpace.VMEM


def _body(partial_ref, scale_ref, out_ref, comm_ref,
          a_tile, b_tile, f_tile,
          rs_send, rs_recv, ag_send, ag_recv, loc_sem):
    my = lax.axis_index("i")
    right = lax.rem(my + 1, N_DEV)
    left = lax.rem(my + N_DEV - 1, N_DEV)
    peers = (right, left)

    barrier_sem = pltpu.get_barrier_semaphore()
    for nbr in peers:
        pl.semaphore_signal(barrier_sem, inc=1, device_id=(nbr,),
                            device_id_type=pl.DeviceIdType.MESH)
    pl.semaphore_wait(barrier_sem, 2)

    def copy(src, dst, sem):
        cp = pltpu.make_async_copy(src, dst, sem)
        cp.start()
        return cp

    def rs_chunks(d, s):
        if d == 0:
            return (lax.rem(my - s + N_DEV, N_DEV),
                    lax.rem(my - s - 1 + N_DEV, N_DEV))
        return lax.rem(my + s, N_DEV), lax.rem(my + s + 1, N_DEV)

    def make_rs(d, s, q):
        col = pl.ds(d * H, H)
        c_send, _ = rs_chunks(d, s)
        if s == 0:
            src = partial_ref.at[pl.ds(c_send * CHUNK + q * TILE, TILE), col]
        else:
            src = comm_ref.at[s - 1, pl.ds(q * TILE, TILE), col]
        return pltpu.make_async_remote_copy(
            src_ref=src,
            dst_ref=comm_ref.at[s, pl.ds(q * TILE, TILE), col],
            send_sem=rs_send.at[d, s, q],
            recv_sem=rs_recv.at[d, s, q],
            device_id=(peers[d],),
            device_id_type=pl.DeviceIdType.MESH,
        )

    def make_ag(d, t, q):
        col = pl.ds(d * H, H)
        if d == 0:
            c = lax.rem(my + 1 - t + N_DEV, N_DEV)
        else:
            c = lax.rem(my - 1 + t + N_DEV, N_DEV)
        rows = pl.ds(c * CHUNK + q * TILE, TILE)
        return pltpu.make_async_remote_copy(
            src_ref=out_ref.at[rows, col],
            dst_ref=out_ref.at[rows, col],
            send_sem=ag_send.at[d, t, q],
            recv_sem=ag_recv.at[d, t, q],
            device_id=(peers[d],),
            device_id_type=pl.DeviceIdType.MESH,
        )

    rs_rdma = {}
    ag_rdma = {}

    for q in range(SUB):
        for d in (0, 1):
            r = make_rs(d, 0, q)
            r.start()
            rs_rdma[(d, 0, q)] = r

    for s in range(N_DEV - 1):
        for q in range(SUB):
            for d in (0, 1):
                rs_rdma[(d, s, q)].wait_recv()
                col = pl.ds(d * H, H)
                row0 = rs_chunks(d, s)[1] * CHUNK + q * TILE
                ca = copy(comm_ref.at[s, pl.ds(q * TILE, TILE), col],
                          a_tile, loc_sem.at[0])
                cb = copy(partial_ref.at[pl.ds(row0, TILE), col],
                          b_tile, loc_sem.at[1])
                ca.wait()
                cb.wait()
                if s < N_DEV - 2:
                    a_tile[...] = a_tile[...] + b_tile[...]
                    copy(a_tile, comm_ref.at[s, pl.ds(q * TILE, TILE), col],
                         loc_sem.at[0]).wait()
                    r = make_rs(d, s + 1, q)
                    r.start()
                    rs_rdma[(d, s + 1, q)] = r
                else:
                    acc = a_tile[...] + b_tile[...]
                    y = acc * scale_ref[0, 0]
                    z = jnp.clip(y, -60.0, 60.0)
                    f_tile[...] = y / (1.0 + jnp.exp(-z))
                    copy(f_tile, out_ref.at[pl.ds(row0, TILE), col],
                         loc_sem.at[0]).wait()
                    r = make_ag(d, 0, q)
                    r.start()
                    ag_rdma[(d, 0, q)] = r

    for t in range(N_DEV - 1):
        for q in range(SUB):
            for d in (0, 1):
                ag_rdma[(d, t, q)].wait_recv()
                if t < N_DEV - 2:
                    r = make_ag(d, t + 1, q)
                    r.start()
                    ag_rdma[(d, t + 1, q)] = r

    for r in rs_rdma.values():
        r.wait_send()
    for r in ag_rdma.values():
        r.wait_send()


def kernel(x, w_mat, scale_x, scale_w):
    partial = lax.dot_general(
        x.astype(jnp.bfloat16), w_mat.astype(jnp.bfloat16),
        dimension_numbers=(((1,), (0,)), ((), ())),
        preferred_element_type=jnp.float32,
    )
    scale = (scale_x[0] * scale_w[0]).reshape(1, 1)

    out, _ = pl.pallas_call(
        _body,
        out_shape=[
            jax.ShapeDtypeStruct((M, N), jnp.float32),
            jax.ShapeDtypeStruct((N_DEV - 1, CHUNK, N), jnp.float32),
        ],
        in_specs=[
            pl.BlockSpec(memory_space=_HBM),
            pl.BlockSpec(memory_space=_SMEM),
        ],
        out_specs=[
            pl.BlockSpec(memory_space=_HBM),
            pl.BlockSpec(memory_space=_HBM),
        ],
        scratch_shapes=[
            _VMEM(shape=(TILE, H), dtype=jnp.float32),
            _VMEM(shape=(TILE, H), dtype=jnp.float32),
            _VMEM(shape=(TILE, H), dtype=jnp.float32),
            pltpu.SemaphoreType.DMA((2, N_DEV - 1, SUB)),
            pltpu.SemaphoreType.DMA((2, N_DEV - 1, SUB)),
            pltpu.SemaphoreType.DMA((2, N_DEV - 1, SUB)),
            pltpu.SemaphoreType.DMA((2, N_DEV - 1, SUB)),
            pltpu.SemaphoreType.DMA((2,)),
        ],
        compiler_params=pltpu.CompilerParams(collective_id=0),
    )(partial, scale)
    return out


# device time: 1189567 ns/iter; 2.0821x vs baseline; 1.0618x over previous
import jax
import jax.numpy as jnp
from jax import lax
from jax.experimental import pallas as pl
from jax.experimental.pallas import tpu as pltpu

N_DEV = 4
M, K_LOC, N = 4096, 1024, 8192
CHUNK = M // N_DEV
H = N // 2
SUB = 4
TILE = CHUNK // SUB

_HBM = pltpu.MemorySpace.HBM
_SMEM = pltpu.MemorySpace.SMEM
_VMEM = pltpu.MemorySpace.VMEM


def _body(x_ref, w_ref, scale_ref, out_ref, comm_ref, partial_ref,
          w_bf, xt_i8, g_tile, a_tile, b_tile,
          rs_send, rs_recv, ag_send, ag_recv, loc_sem):
    my = lax.axis_index("i")
    right = lax.rem(my + 1, N_DEV)
    left = lax.rem(my + N_DEV - 1, N_DEV)
    peers = (right, left)

    barrier_sem = pltpu.get_barrier_semaphore()
    for nbr in peers:
        pl.semaphore_signal(barrier_sem, inc=1, device_id=(nbr,),
                            device_id_type=pl.DeviceIdType.MESH)
    pl.semaphore_wait(barrier_sem, 2)

    def copy(src, dst, sem):
        cp = pltpu.make_async_copy(src, dst, sem)
        cp.start()
        return cp

    def rs_chunks(d, s):
        if d == 0:
            return (lax.rem(my - s + N_DEV, N_DEV),
                    lax.rem(my - s - 1 + N_DEV, N_DEV))
        return lax.rem(my + s, N_DEV), lax.rem(my + s + 1, N_DEV)

    def make_rs(d, s, q):
        col = pl.ds(d * H, H)
        c_send, _ = rs_chunks(d, s)
        if s == 0:
            src = partial_ref.at[pl.ds(c_send * CHUNK + q * TILE, TILE), col]
        else:
            src = comm_ref.at[s - 1, pl.ds(q * TILE, TILE), col]
        return pltpu.make_async_remote_copy(
            src_ref=src,
            dst_ref=comm_ref.at[s, pl.ds(q * TILE, TILE), col],
            send_sem=rs_send.at[d, s, q],
            recv_sem=rs_recv.at[d, s, q],
            device_id=(peers[d],),
            device_id_type=pl.DeviceIdType.MESH,
        )

    def make_ag(d, t, q):
        col = pl.ds(d * H, H)
        if d == 0:
            c = lax.rem(my + 1 - t + N_DEV, N_DEV)
        else:
            c = lax.rem(my - 1 + t + N_DEV, N_DEV)
        rows = pl.ds(c * CHUNK + q * TILE, TILE)
        return pltpu.make_async_remote_copy(
            src_ref=out_ref.at[rows, col],
            dst_ref=out_ref.at[rows, col],
            send_sem=ag_send.at[d, t, q],
            recv_sem=ag_recv.at[d, t, q],
            device_id=(peers[d],),
            device_id_type=pl.DeviceIdType.MESH,
        )

    rs_rdma = {}
    ag_rdma = {}

    copy(w_ref, w_bf, loc_sem.at[0]).wait()

    def gemm_tile(c, q):
        row = c * CHUNK + q * TILE
        copy(x_ref.at[pl.ds(row, TILE), :], xt_i8, loc_sem.at[0]).wait()
        xv = xt_i8[...].astype(jnp.bfloat16)
        for dd in (0, 1):
            g_tile[...] = lax.dot_general(
                xv, w_bf[:, pl.ds(dd * H, H)],
                dimension_numbers=(((1,), (0,)), ((), ())),
                preferred_element_type=jnp.float32,
            )
            copy(g_tile,
                 partial_ref.at[pl.ds(row, TILE), pl.ds(dd * H, H)],
                 loc_sem.at[1]).wait()

    for q in range(SUB):
        gemm_tile(my, q)
        for d in (0, 1):
            r = make_rs(d, 0, q)
            r.start()
            rs_rdma[(d, 0, q)] = r

    for c in (left, right, lax.rem(my + 2, N_DEV)):
        for q in range(SUB):
            gemm_tile(c, q)

    for s in range(N_DEV - 1):
        for q in range(SUB):
            for d in (0, 1):
                rs_rdma[(d, s, q)].wait_recv()
                col = pl.ds(d * H, H)
                row0 = rs_chunks(d, s)[1] * CHUNK + q * TILE
                ca = copy(comm_ref.at[s, pl.ds(q * TILE, TILE), col],
                          a_tile, loc_sem.at[0])
                cb = copy(partial_ref.at[pl.ds(row0, TILE), col],
                          b_tile, loc_sem.at[1])
                ca.wait()
                cb.wait()
                if s < N_DEV - 2:
                    a_tile[...] = a_tile[...] + b_tile[...]
                    copy(a_tile, comm_ref.at[s, pl.ds(q * TILE, TILE), col],
                         loc_sem.at[0]).wait()
                    r = make_rs(d, s + 1, q)
                    r.start()
                    rs_rdma[(d, s + 1, q)] = r
                else:
                    acc = a_tile[...] + b_tile[...]
                    y = acc * scale_ref[0, 0]
                    z = jnp.clip(y, -60.0, 60.0)
                    g_tile[...] = y / (1.0 + jnp.exp(-z))
                    copy(g_tile, out_ref.at[pl.ds(row0, TILE), col],
                         loc_sem.at[0]).wait()
                    r = make_ag(d, 0, q)
                    r.start()
                    ag_rdma[(d, 0, q)] = r

    for t in range(N_DEV - 1):
        for q in range(SUB):
            for d in (0, 1):
                ag_rdma[(d, t, q)].wait_recv()
                if t < N_DEV - 2:
                    r = make_ag(d, t + 1, q)
                    r.start()
                    ag_rdma[(d, t + 1, q)] = r

    for r in rs_rdma.values():
        r.wait_send()
    for r in ag_rdma.values():
        r.wait_send()


def kernel(x, w_mat, scale_x, scale_w):
    scale = (scale_x[0] * scale_w[0]).reshape(1, 1)

    out, _, _ = pl.pallas_call(
        _body,
        out_shape=[
            jax.ShapeDtypeStruct((M, N), jnp.float32),
            jax.ShapeDtypeStruct((N_DEV - 1, CHUNK, N), jnp.float32),
            jax.ShapeDtypeStruct((M, N), jnp.float32),
        ],
        in_specs=[
            pl.BlockSpec(memory_space=_HBM),
            pl.BlockSpec(memory_space=_HBM),
            pl.BlockSpec(memory_space=_SMEM),
        ],
        out_specs=[
            pl.BlockSpec(memory_space=_HBM),
            pl.BlockSpec(memory_space=_HBM),
            pl.BlockSpec(memory_space=_HBM),
        ],
        scratch_shapes=[
            _VMEM(shape=(K_LOC, N), dtype=jnp.bfloat16),
            _VMEM(shape=(TILE, K_LOC), dtype=jnp.int8),
            _VMEM(shape=(TILE, H), dtype=jnp.float32),
            _VMEM(shape=(TILE, H), dtype=jnp.float32),
            _VMEM(shape=(TILE, H), dtype=jnp.float32),
            pltpu.SemaphoreType.DMA((2, N_DEV - 1, SUB)),
            pltpu.SemaphoreType.DMA((2, N_DEV - 1, SUB)),
            pltpu.SemaphoreType.DMA((2, N_DEV - 1, SUB)),
            pltpu.SemaphoreType.DMA((2, N_DEV - 1, SUB)),
            pltpu.SemaphoreType.DMA((2,)),
        ],
        compiler_params=pltpu.CompilerParams(collective_id=0),
    )(x, w_mat.astype(jnp.bfloat16), scale)
    return out


# device time: 1188637 ns/iter; 2.0837x vs baseline; 1.0008x over previous
import jax
import jax.numpy as jnp
from jax import lax
from jax.experimental import pallas as pl
from jax.experimental.pallas import tpu as pltpu

N_DEV = 4
M, K_LOC, N = 4096, 1024, 8192
CHUNK = M // N_DEV
H = N // 2
SUB = 4
TILE = CHUNK // SUB

_HBM = pltpu.MemorySpace.HBM
_SMEM = pltpu.MemorySpace.SMEM
_VMEM = pltpu.MemorySpace.VMEM


def _body(x_ref, w_ref, scale_ref, out_ref, comm_ref, partial_ref,
          w_bf, xt_i8, g_tile, a_tile, b_tile,
          rs_send, rs_recv, ag_send, ag_recv, loc_sem):
    my = lax.axis_index("i")
    right = lax.rem(my + 1, N_DEV)
    left = lax.rem(my + N_DEV - 1, N_DEV)
    peers = (right, left)

    barrier_sem = pltpu.get_barrier_semaphore()
    for nbr in peers:
        pl.semaphore_signal(barrier_sem, inc=1, device_id=(nbr,),
                            device_id_type=pl.DeviceIdType.MESH)
    pl.semaphore_wait(barrier_sem, 2)

    def copy(src, dst, sem):
        cp = pltpu.make_async_copy(src, dst, sem)
        cp.start()
        return cp

    def rs_chunks(d, s):
        if d == 0:
            return (lax.rem(my - s + N_DEV, N_DEV),
                    lax.rem(my - s - 1 + N_DEV, N_DEV))
        return lax.rem(my + s, N_DEV), lax.rem(my + s + 1, N_DEV)

    def make_rs(d, s, q):
        col = pl.ds(d * H, H)
        c_send, _ = rs_chunks(d, s)
        if s == 0:
            src = partial_ref.at[pl.ds(c_send * CHUNK + q * TILE, TILE), col]
        else:
            src = comm_ref.at[s - 1, pl.ds(q * TILE, TILE), col]
        return pltpu.make_async_remote_copy(
            src_ref=src,
            dst_ref=comm_ref.at[s, pl.ds(q * TILE, TILE), col],
            send_sem=rs_send.at[d, s, q],
            recv_sem=rs_recv.at[d, s, q],
            device_id=(peers[d],),
            device_id_type=pl.DeviceIdType.MESH,
        )

    def make_ag(d, t, q):
        col = pl.ds(d * H, H)
        if d == 0:
            c = lax.rem(my + 1 - t + N_DEV, N_DEV)
        else:
            c = lax.rem(my - 1 + t + N_DEV, N_DEV)
        rows = pl.ds(c * CHUNK + q * TILE, TILE)
        return pltpu.make_async_remote_copy(
            src_ref=out_ref.at[rows, col],
            dst_ref=out_ref.at[rows, col],
            send_sem=ag_send.at[d, t, q],
            recv_sem=ag_recv.at[d, t, q],
            device_id=(peers[d],),
            device_id_type=pl.DeviceIdType.MESH,
        )

    rs_rdma = {}
    ag_rdma = {}

    copy(w_ref, w_bf, loc_sem.at[0]).wait()

    def gemm_tile(c, q):
        row = c * CHUNK + q * TILE
        copy(x_ref.at[pl.ds(row, TILE), :], xt_i8, loc_sem.at[0]).wait()
        xv = xt_i8[...].astype(jnp.bfloat16)
        for dd in (0, 1):
            g_tile[...] = lax.dot_general(
                xv, w_bf[:, pl.ds(dd * H, H)],
                dimension_numbers=(((1,), (0,)), ((), ())),
                preferred_element_type=jnp.float32,
            )
            copy(g_tile,
                 partial_ref.at[pl.ds(row, TILE), pl.ds(dd * H, H)],
                 loc_sem.at[1]).wait()

    for q in range(SUB):
        gemm_tile(my, q)
        for d in (0, 1):
            r = make_rs(d, 0, q)
            r.start()
            rs_rdma[(d, 0, q)] = r

    for c in (left, right):
        for q in range(SUB):
            gemm_tile(c, q)

    for s in range(N_DEV - 1):
        if s == 1:
            for q in range(SUB):
                gemm_tile(lax.rem(my + 2, N_DEV), q)
        for q in range(SUB):
            for d in (0, 1):
                rs_rdma[(d, s, q)].wait_recv()
                col = pl.ds(d * H, H)
                row0 = rs_chunks(d, s)[1] * CHUNK + q * TILE
                ca = copy(comm_ref.at[s, pl.ds(q * TILE, TILE), col],
                          a_tile, loc_sem.at[0])
                cb = copy(partial_ref.at[pl.ds(row0, TILE), col],
                          b_tile, loc_sem.at[1])
                ca.wait()
                cb.wait()
                if s < N_DEV - 2:
                    a_tile[...] = a_tile[...] + b_tile[...]
                    copy(a_tile, comm_ref.at[s, pl.ds(q * TILE, TILE), col],
                         loc_sem.at[0]).wait()
                    r = make_rs(d, s + 1, q)
                    r.start()
                    rs_rdma[(d, s + 1, q)] = r
                else:
                    acc = a_tile[...] + b_tile[...]
                    y = acc * scale_ref[0, 0]
                    z = jnp.clip(y, -60.0, 60.0)
                    g_tile[...] = y / (1.0 + jnp.exp(-z))
                    copy(g_tile, out_ref.at[pl.ds(row0, TILE), col],
                         loc_sem.at[0]).wait()
                    r = make_ag(d, 0, q)
                    r.start()
                    ag_rdma[(d, 0, q)] = r

    for t in range(N_DEV - 1):
        for q in range(SUB):
            for d in (0, 1):
                ag_rdma[(d, t, q)].wait_recv()
                if t < N_DEV - 2:
                    r = make_ag(d, t + 1, q)
                    r.start()
                    ag_rdma[(d, t + 1, q)] = r

    for r in rs_rdma.values():
        r.wait_send()
    for r in ag_rdma.values():
        r.wait_send()


def kernel(x, w_mat, scale_x, scale_w):
    scale = (scale_x[0] * scale_w[0]).reshape(1, 1)

    out, _, _ = pl.pallas_call(
        _body,
        out_shape=[
            jax.ShapeDtypeStruct((M, N), jnp.float32),
            jax.ShapeDtypeStruct((N_DEV - 1, CHUNK, N), jnp.float32),
            jax.ShapeDtypeStruct((M, N), jnp.float32),
        ],
        in_specs=[
            pl.BlockSpec(memory_space=_HBM),
            pl.BlockSpec(memory_space=_HBM),
            pl.BlockSpec(memory_space=_SMEM),
        ],
        out_specs=[
            pl.BlockSpec(memory_space=_HBM),
            pl.BlockSpec(memory_space=_HBM),
            pl.BlockSpec(memory_space=_HBM),
        ],
        scratch_shapes=[
            _VMEM(shape=(K_LOC, N), dtype=jnp.bfloat16),
            _VMEM(shape=(TILE, K_LOC), dtype=jnp.int8),
            _VMEM(shape=(TILE, H), dtype=jnp.float32),
            _VMEM(shape=(TILE, H), dtype=jnp.float32),
            _VMEM(shape=(TILE, H), dtype=jnp.float32),
            pltpu.SemaphoreType.DMA((2, N_DEV - 1, SUB)),
            pltpu.SemaphoreType.DMA((2, N_DEV - 1, SUB)),
            pltpu.SemaphoreType.DMA((2, N_DEV - 1, SUB)),
            pltpu.SemaphoreType.DMA((2, N_DEV - 1, SUB)),
            pltpu.SemaphoreType.DMA((2,)),
        ],
        compiler_params=pltpu.CompilerParams(collective_id=0),
    )(x, w_mat.astype(jnp.bfloat16), scale)
    return out
